# initial kernel scaffold (unmeasured)
import jax
import jax.numpy as jnp
from jax import lax
from jax.experimental import pallas as pl
from jax.experimental.pallas import tpu as pltpu

N_DEV = 4


def kernel(x, w_mat, scale_x, scale_w):
    m_full, k_sh = x.shape
    k_full, n_full = w_mat.shape
    m_blk = m_full // N_DEV
    n_blk = 1024
    n_chunks = n_full // n_blk
    n_mm = n_chunks * N_DEV

    def body(x_ref, w_ref, sx_ref, sw_ref, out_ref,
             comm_ref, w_buf, send_sems, recv_sems, w_sems, local_sem):
        me = lax.axis_index("i")

        barrier = pltpu.get_barrier_semaphore()
        for d in range(1, N_DEV):
            t = (me + d) % N_DEV
            pl.semaphore_signal(barrier, inc=1, device_id=(t,),
                                device_id_type=pl.DeviceIdType.MESH)
        pl.semaphore_wait(barrier, N_DEV - 1)

        lcopy = pltpu.make_async_copy(
            x_ref.at[pl.ds(me * m_blk, m_blk), :], comm_ref.at[me], local_sem)
        lcopy.start()

        def send_desc(d):
            t = (me + d) % N_DEV
            return pltpu.make_async_remote_copy(
                src_ref=x_ref.at[pl.ds(t * m_blk, m_blk), :],
                dst_ref=comm_ref.at[me],
                send_sem=send_sems.at[d],
                recv_sem=recv_sems.at[me],
                device_id=(t,),
                device_id_type=pl.DeviceIdType.MESH,
            )

        for d in range(1, N_DEV):
            send_desc(d).start()

        def w_dma(c, slot):
            n, j = c // N_DEV, c % N_DEV
            return pltpu.make_async_copy(
                w_ref.at[pl.ds(j * k_sh, k_sh), pl.ds(n * n_blk, n_blk)],
                w_buf.at[slot], w_sems.at[slot])

        w_dma(0, 0).start()

        lcopy.wait()
        for d in range(1, N_DEV):
            j = (me + d) % N_DEV
            recv = pltpu.make_async_remote_copy(
                src_ref=x_ref.at[pl.ds(0, m_blk), :],
                dst_ref=comm_ref.at[j],
                send_sem=send_sems.at[d],
                recv_sem=recv_sems.at[j],
                device_id=(me,),
                device_id_type=pl.DeviceIdType.MESH,
            )
            recv.wait_recv()

        s = sx_ref[0] * sw_ref[0]

        for c in range(n_mm):
            n, j = c // N_DEV, c % N_DEV
            slot = c % 2
            if c + 1 < n_mm:
                w_dma(c + 1, (c + 1) % 2).start()
            w_dma(c, slot).wait()
            contrib = jnp.dot(
                comm_ref[j].astype(jnp.bfloat16),
                w_buf[slot].astype(jnp.bfloat16),
                preferred_element_type=jnp.float32,
            )
            nsl = pl.ds(n * n_blk, n_blk)
            if j == 0:
                out_ref[:, nsl] = contrib
            elif j == N_DEV - 1:
                out_ref[:, nsl] = (out_ref[:, nsl] + contrib) * s
            else:
                out_ref[:, nsl] = out_ref[:, nsl] + contrib

        for d in range(1, N_DEV):
            send_desc(d).wait_send()

    return pl.pallas_call(
        body,
        out_shape=jax.ShapeDtypeStruct((m_blk, n_full), jnp.float32),
        in_specs=[
            pl.BlockSpec(memory_space=pltpu.ANY),
            pl.BlockSpec(memory_space=pltpu.ANY),
            pl.BlockSpec(memory_space=pltpu.SMEM),
            pl.BlockSpec(memory_space=pltpu.SMEM),
        ],
        out_specs=pl.BlockSpec(memory_space=pltpu.VMEM),
        scratch_shapes=[
            pltpu.VMEM((N_DEV, m_blk, k_sh), x.dtype),
            pltpu.VMEM((2, k_sh, n_blk), w_mat.dtype),
            pltpu.SemaphoreType.DMA((N_DEV,)),
            pltpu.SemaphoreType.DMA((N_DEV,)),
            pltpu.SemaphoreType.DMA((2,)),
            pltpu.SemaphoreType.DMA,
        ],
        compiler_params=pltpu.CompilerParams(collective_id=0),
    )(x, w_mat, scale_x, scale_w)


# baseline (device time: 207738 ns/iter reference)
import jax
import jax.numpy as jnp
from jax import lax
from jax.experimental import pallas as pl
from jax.experimental.pallas import tpu as pltpu

N_DEV = 4


def kernel(x, w_mat, scale_x, scale_w):
    m_full, k_sh = x.shape
    k_full, n_full = w_mat.shape
    m_blk = m_full // N_DEV
    n_blk = 1024
    n_chunks = n_full // n_blk
    n_mm = n_chunks * N_DEV

    def body(x_ref, w_ref, sx_ref, sw_ref, out_ref,
             comm_ref, w_buf, send_sems, recv_sems, w_sems, local_sem):
        me = lax.axis_index("i")

        barrier = pltpu.get_barrier_semaphore()
        for d in range(1, N_DEV):
            t = (me + d) % N_DEV
            pl.semaphore_signal(barrier, inc=1, device_id=(t,),
                                device_id_type=pl.DeviceIdType.MESH)
        pl.semaphore_wait(barrier, N_DEV - 1)

        lcopy = pltpu.make_async_copy(
            x_ref.at[pl.ds(me * m_blk, m_blk), :], comm_ref.at[me], local_sem)
        lcopy.start()

        def send_desc(d):
            t = (me + d) % N_DEV
            return pltpu.make_async_remote_copy(
                src_ref=x_ref.at[pl.ds(t * m_blk, m_blk), :],
                dst_ref=comm_ref.at[me],
                send_sem=send_sems.at[d],
                recv_sem=recv_sems.at[me],
                device_id=(t,),
                device_id_type=pl.DeviceIdType.MESH,
            )

        for d in range(1, N_DEV):
            send_desc(d).start()

        def w_dma(c, slot):
            n, j = c // N_DEV, c % N_DEV
            return pltpu.make_async_copy(
                w_ref.at[pl.ds(j * k_sh, k_sh), pl.ds(n * n_blk, n_blk)],
                w_buf.at[slot], w_sems.at[slot])

        w_dma(0, 0).start()

        lcopy.wait()
        for d in range(1, N_DEV):
            j = (me + d) % N_DEV
            recv = pltpu.make_async_remote_copy(
                src_ref=x_ref.at[pl.ds(0, m_blk), :],
                dst_ref=comm_ref.at[j],
                send_sem=send_sems.at[d],
                recv_sem=recv_sems.at[j],
                device_id=(me,),
                device_id_type=pl.DeviceIdType.MESH,
            )
            recv.wait_recv()

        s = sx_ref[0] * sw_ref[0]

        for c in range(n_mm):
            n, j = c // N_DEV, c % N_DEV
            slot = c % 2
            if c + 1 < n_mm:
                w_dma(c + 1, (c + 1) % 2).start()
            w_dma(c, slot).wait()
            contrib = jnp.dot(
                comm_ref[j].astype(jnp.bfloat16),
                w_buf[slot].astype(jnp.bfloat16),
                preferred_element_type=jnp.float32,
            )
            nsl = pl.ds(n * n_blk, n_blk)
            if j == 0:
                out_ref[:, nsl] = contrib
            elif j == N_DEV - 1:
                out_ref[:, nsl] = (out_ref[:, nsl] + contrib) * s
            else:
                out_ref[:, nsl] = out_ref[:, nsl] + contrib

        for d in range(1, N_DEV):
            send_desc(d).wait_send()

    return pl.pallas_call(
        body,
        out_shape=jax.ShapeDtypeStruct((m_blk, n_full), jnp.float32),
        in_specs=[
            pl.BlockSpec(memory_space=pl.ANY),
            pl.BlockSpec(memory_space=pl.ANY),
            pl.BlockSpec(memory_space=pltpu.SMEM),
            pl.BlockSpec(memory_space=pltpu.SMEM),
        ],
        out_specs=pl.BlockSpec(memory_space=pltpu.VMEM),
        scratch_shapes=[
            pltpu.VMEM((N_DEV, m_blk, k_sh), x.dtype),
            pltpu.VMEM((2, k_sh, n_blk), w_mat.dtype),
            pltpu.SemaphoreType.DMA((N_DEV,)),
            pltpu.SemaphoreType.DMA((N_DEV,)),
            pltpu.SemaphoreType.DMA((2,)),
            pltpu.SemaphoreType.DMA,
        ],
        compiler_params=pltpu.CompilerParams(
            collective_id=0, vmem_limit_bytes=64 * 1024 * 1024),
    )(x, w_mat, scale_x, scale_w)


# device time: 100758 ns/iter; 2.0618x vs baseline; 2.0618x over previous
import jax
import jax.numpy as jnp
from jax import lax
from jax.experimental import pallas as pl
from jax.experimental.pallas import tpu as pltpu

N_DEV = 4


def kernel(x, w_mat, scale_x, scale_w):
    m_full, k_sh = x.shape
    k_full, n_full = w_mat.shape
    m_blk = m_full // N_DEV
    n_blk = 1024
    n_chunks = n_full // n_blk
    n_mm = n_chunks * N_DEV
    SEND_ORDER = (1, 3, 2)
    COMPUTE_ORDER = (0, 1, 3, 2)

    def body(x_ref, w_ref, sx_ref, sw_ref, out_ref,
             xs, x8, comm_ref, w_buf, acc_ref,
             xs_sems, send_sems, recv_sems, w_sems, out_sems):
        me = lax.axis_index("i")

        barrier = pltpu.get_barrier_semaphore()
        for d in range(1, N_DEV):
            t = (me + d) % N_DEV
            pl.semaphore_signal(barrier, inc=1, device_id=(t,),
                                device_id_type=pl.DeviceIdType.MESH)
        pl.semaphore_wait(barrier, N_DEV - 1)

        def w_dma(c, slot):
            si, n = c // n_chunks, c % n_chunks
            j = (me + COMPUTE_ORDER[si]) % N_DEV
            return pltpu.make_async_copy(
                w_ref.at[pl.ds(j * k_sh, k_sh), pl.ds(n * n_blk, n_blk)],
                w_buf.at[slot], w_sems.at[slot])

        w_dma(0, 0).start()

        blocks = SEND_ORDER + (0,)
        def xs_dma(k):
            t = (me + blocks[k]) % N_DEV
            return pltpu.make_async_copy(
                x_ref.at[pl.ds(t * m_blk, m_blk), :], xs.at[k % 2],
                xs_sems.at[k % 2])

        def send_desc(k):
            d = blocks[k]
            t = (me + d) % N_DEV
            return pltpu.make_async_remote_copy(
                src_ref=x8.at[k],
                dst_ref=comm_ref.at[me],
                send_sem=send_sems.at[d],
                recv_sem=recv_sems.at[me],
                device_id=(t,),
                device_id_type=pl.DeviceIdType.MESH,
            )

        xs_dma(0).start()
        for k, d in enumerate(blocks):
            if k + 1 < len(blocks):
                xs_dma(k + 1).start()
            xs_dma(k).wait()
            x8[k] = xs[k % 2].astype(jnp.float8_e4m3fn)
            if d != 0:
                send_desc(k).start()

        s = sx_ref[0] * sw_ref[0]

        for si, d in enumerate(COMPUTE_ORDER):
            j = (me + d) % N_DEV
            if si > 0:
                pltpu.make_async_remote_copy(
                    src_ref=x8.at[0],
                    dst_ref=comm_ref.at[j],
                    send_sem=send_sems.at[d],
                    recv_sem=recv_sems.at[j],
                    device_id=(me,),
                    device_id_type=pl.DeviceIdType.MESH,
                ).wait_recv()
            for n in range(n_chunks):
                c = si * n_chunks + n
                slot = c % 2
                if c + 1 < n_mm:
                    w_dma(c + 1, (c + 1) % 2).start()
                w_dma(c, slot).wait()
                a8 = x8[len(blocks) - 1] if si == 0 else comm_ref[j]
                contrib = jnp.dot(
                    a8,
                    w_buf[slot].astype(jnp.float8_e5m2),
                    preferred_element_type=jnp.float32,
                )
                nsl = pl.ds(n * n_blk, n_blk)
                if si == 0:
                    acc_ref[:, nsl] = contrib
                elif si < N_DEV - 1:
                    acc_ref[:, nsl] = acc_ref[:, nsl] + contrib
                else:
                    acc_ref[:, nsl] = (acc_ref[:, nsl] + contrib) * s
                    pltpu.make_async_copy(
                        acc_ref.at[:, nsl], out_ref.at[:, nsl],
                        out_sems.at[n]).start()

        for n in range(n_chunks):
            nsl = pl.ds(n * n_blk, n_blk)
            pltpu.make_async_copy(
                acc_ref.at[:, nsl], out_ref.at[:, nsl], out_sems.at[n]).wait()
        for k in range(len(SEND_ORDER)):
            send_desc(k).wait_send()

    return pl.pallas_call(
        body,
        out_shape=jax.ShapeDtypeStruct((m_blk, n_full), jnp.float32),
        in_specs=[
            pl.BlockSpec(memory_space=pl.ANY),
            pl.BlockSpec(memory_space=pl.ANY),
            pl.BlockSpec(memory_space=pltpu.SMEM),
            pl.BlockSpec(memory_space=pltpu.SMEM),
        ],
        out_specs=pl.BlockSpec(memory_space=pl.ANY),
        scratch_shapes=[
            pltpu.VMEM((2, m_blk, k_sh), x.dtype),
            pltpu.VMEM((N_DEV, m_blk, k_sh), jnp.float8_e4m3fn),
            pltpu.VMEM((N_DEV, m_blk, k_sh), jnp.float8_e4m3fn),
            pltpu.VMEM((2, k_sh, n_blk), w_mat.dtype),
            pltpu.VMEM((m_blk, n_full), jnp.float32),
            pltpu.SemaphoreType.DMA((2,)),
            pltpu.SemaphoreType.DMA((N_DEV,)),
            pltpu.SemaphoreType.DMA((N_DEV,)),
            pltpu.SemaphoreType.DMA((2,)),
            pltpu.SemaphoreType.DMA((n_chunks,)),
        ],
        compiler_params=pltpu.CompilerParams(
            collective_id=0, vmem_limit_bytes=64 * 1024 * 1024),
    )(x, w_mat, scale_x, scale_w)


# device time: 86877 ns/iter; 2.3912x vs baseline; 1.1598x over previous
import jax
import jax.numpy as jnp
from jax import lax
from jax.experimental import pallas as pl
from jax.experimental.pallas import tpu as pltpu

N_DEV = 4


def kernel(x, w_mat, scale_x, scale_w):
    m_full, k_sh = x.shape
    k_full, n_full = w_mat.shape
    m_blk = m_full // N_DEV
    n_blk = 1024
    n_chunks = n_full // n_blk
    n_mm = n_chunks * N_DEV
    W_SLOTS, W_DEPTH = 3, 2
    SEND_ORDER = (1, 3, 2)
    COMPUTE_ORDER = ((None, 0), (2, 1), (0, 3), (1, 2))

    def body(x_ref, w_ref, sx_ref, sw_ref, out_ref,
             xs, x8, comm_ref, w_buf, acc_ref,
             xs_sems, send_sems, recv_sems, w_sems, out_sems):
        me = lax.axis_index("i")

        barrier = pltpu.get_barrier_semaphore()
        for d in SEND_ORDER:
            t = (me + d) % N_DEV
            pl.semaphore_signal(barrier, inc=1, device_id=(t,),
                                device_id_type=pl.DeviceIdType.MESH)
        pl.semaphore_wait(barrier, N_DEV - 1)

        def w_dma(c):
            si, n = c // n_chunks, c % n_chunks
            o = COMPUTE_ORDER[si][1]
            j = (me - o) % N_DEV
            slot = c % W_SLOTS
            return pltpu.make_async_copy(
                w_ref.at[pl.ds(j * k_sh, k_sh), pl.ds(n * n_blk, n_blk)],
                w_buf.at[slot], w_sems.at[slot])

        for c in range(W_DEPTH):
            w_dma(c).start()

        blocks = SEND_ORDER + (0,)
        def xs_dma(k):
            t = (me + blocks[k]) % N_DEV
            return pltpu.make_async_copy(
                x_ref.at[pl.ds(t * m_blk, m_blk), :], xs.at[k % 2],
                xs_sems.at[k % 2])

        def send_desc(k):
            d = blocks[k]
            t = (me + d) % N_DEV
            return pltpu.make_async_remote_copy(
                src_ref=x8.at[k],
                dst_ref=comm_ref.at[3 - d],
                send_sem=send_sems.at[d],
                recv_sem=recv_sems.at[3 - d],
                device_id=(t,),
                device_id_type=pl.DeviceIdType.MESH,
            )

        xs_dma(0).start()
        for k, d in enumerate(blocks):
            if k + 1 < len(blocks):
                xs_dma(k + 1).start()
            xs_dma(k).wait()
            x8[k] = xs[k % 2].astype(jnp.float8_e4m3fn)
            if d != 0:
                send_desc(k).start()

        s = sx_ref[0] * sw_ref[0]

        for si, (slot_in, o) in enumerate(COMPUTE_ORDER):
            if slot_in is not None:
                pltpu.make_async_remote_copy(
                    src_ref=x8.at[0],
                    dst_ref=comm_ref.at[slot_in],
                    send_sem=send_sems.at[0],
                    recv_sem=recv_sems.at[slot_in],
                    device_id=(me,),
                    device_id_type=pl.DeviceIdType.MESH,
                ).wait_recv()
            a8 = x8[len(blocks) - 1] if slot_in is None else comm_ref[slot_in]
            for n in range(n_chunks):
                c = si * n_chunks + n
                if c + W_DEPTH < n_mm:
                    w_dma(c + W_DEPTH).start()
                w_dma(c).wait()
                contrib = jnp.dot(
                    a8,
                    w_buf[c % W_SLOTS].astype(jnp.float8_e5m2),
                    preferred_element_type=jnp.float32,
                )
                nsl = pl.ds(n * n_blk, n_blk)
                if si == 0:
                    acc_ref[:, nsl] = contrib
                elif si < N_DEV - 1:
                    acc_ref[:, nsl] = acc_ref[:, nsl] + contrib
                else:
                    acc_ref[:, nsl] = (acc_ref[:, nsl] + contrib) * s
                    pltpu.make_async_copy(
                        acc_ref.at[:, nsl], out_ref.at[:, nsl],
                        out_sems.at[n]).start()

        for n in range(n_chunks):
            nsl = pl.ds(n * n_blk, n_blk)
            pltpu.make_async_copy(
                acc_ref.at[:, nsl], out_ref.at[:, nsl], out_sems.at[n]).wait()
        for k in range(len(SEND_ORDER)):
            send_desc(k).wait_send()

    return pl.pallas_call(
        body,
        out_shape=jax.ShapeDtypeStruct((m_blk, n_full), jnp.float32),
        in_specs=[
            pl.BlockSpec(memory_space=pl.ANY),
            pl.BlockSpec(memory_space=pl.ANY),
            pl.BlockSpec(memory_space=pltpu.SMEM),
            pl.BlockSpec(memory_space=pltpu.SMEM),
        ],
        out_specs=pl.BlockSpec(memory_space=pl.ANY),
        scratch_shapes=[
            pltpu.VMEM((2, m_blk, k_sh), x.dtype),
            pltpu.VMEM((N_DEV, m_blk, k_sh), jnp.float8_e4m3fn),
            pltpu.VMEM((N_DEV - 1, m_blk, k_sh), jnp.float8_e4m3fn),
            pltpu.VMEM((W_SLOTS, k_sh, n_blk), w_mat.dtype),
            pltpu.VMEM((m_blk, n_full), jnp.float32),
            pltpu.SemaphoreType.DMA((2,)),
            pltpu.SemaphoreType.DMA((N_DEV,)),
            pltpu.SemaphoreType.DMA((N_DEV - 1,)),
            pltpu.SemaphoreType.DMA((W_SLOTS,)),
            pltpu.SemaphoreType.DMA((n_chunks,)),
        ],
        compiler_params=pltpu.CompilerParams(
            collective_id=0, vmem_limit_bytes=64 * 1024 * 1024),
    )(x, w_mat, scale_x, scale_w)
